# Initial kernel scaffold; baseline (speedup 1.0000x reference)
import jax
import jax.numpy as jnp
from jax.experimental import pallas as pl


def kernel(x, y, embed, W, b):
    raise NotImplementedError("write your pallas kernel here")


# speedup vs baseline: 1.5031x; 1.5031x over previous
import functools

import jax
import jax.numpy as jnp
from jax import lax
from jax.experimental import pallas as pl
from jax.experimental.pallas import tpu as pltpu
from jax.experimental.pallas import tpu_sc as plsc

V = 100000
D = 256
C = 128
B = 4096
LX = 200
LY = 50

NC = 2
NS = 16
NW = NC * NS
EPW = B // NW
IPE = LX + LY + 6
GCH = 128

MM_BLK = 1000
N_MM = (2 * V) // MM_BLK
NR = (N_MM + 1) * MM_BLK
BIAS_ROW = 2 * V
ZERO_ROW = 2 * V + 1


def _table_body(e_ref, w_ref, b_ref, o_ref):
    g = pl.program_id(0)

    @pl.when(g < N_MM)
    def _():
        o_ref[...] = jnp.dot(
            e_ref[...], w_ref[0],
            preferred_element_type=jnp.float32,
            precision=lax.Precision.HIGHEST,
        )

    @pl.when(g == N_MM)
    def _():
        o_ref[...] = jnp.zeros_like(o_ref)
        o_ref[0:1, :] = b_ref[...]


def _build_table(embed, wstack, bias2d):
    return pl.pallas_call(
        _table_body,
        grid=(N_MM + 1,),
        in_specs=[
            pl.BlockSpec((MM_BLK, D), lambda g: (g % (N_MM // 2), 0)),
            pl.BlockSpec((1, D, C), lambda g: (jnp.minimum(g // (N_MM // 2), 1), 0, 0)),
            pl.BlockSpec((1, C), lambda g: (0, 0)),
        ],
        out_specs=pl.BlockSpec((MM_BLK, C), lambda g: (g, 0)),
        out_shape=jax.ShapeDtypeStruct((NR, C), jnp.float32),
    )(embed, wstack, bias2d)


def _sum_rows(rows_ref, accs):
    def body(r, accs):
        return tuple(accs[j] + rows_ref[r, pl.ds(16 * j, 16)] for j in range(C // 16))
    return lax.fori_loop(0, GCH, body, accs)


@functools.partial(
    pl.kernel,
    mesh=plsc.VectorSubcoreMesh(core_axis_name="c", subcore_axis_name="s"),
    out_type=jax.ShapeDtypeStruct((B * C,), jnp.float32),
    scratch_types=[
        pltpu.VMEM((EPW * IPE,), jnp.int32),
        pltpu.VMEM((GCH, C), jnp.float32),
        pltpu.VMEM((GCH, C), jnp.float32),
        pltpu.VMEM((EPW * C,), jnp.float32),
        pltpu.SemaphoreType.DMA,
        pltpu.SemaphoreType.DMA,
    ],
)
def _sc_pool(table_hbm, idx_hbm, out_hbm, idxv, rows_a, rows_b, outv, sem_a, sem_b):
    wid = lax.axis_index("s") * NC + lax.axis_index("c")
    ibase = wid * (EPW * IPE)
    pltpu.sync_copy(idx_hbm.at[pl.ds(ibase, EPW * IPE)], idxv)

    @pl.loop(0, EPW)
    def _(e):
        off = e * IPE
        cp_a = pltpu.async_copy(
            table_hbm.at[idxv.at[pl.ds(off, GCH)]], rows_a, sem_a)
        cp_b = pltpu.async_copy(
            table_hbm.at[idxv.at[pl.ds(off + GCH, GCH)]], rows_b, sem_b)
        zeros = tuple(jnp.zeros((16,), jnp.float32) for _ in range(C // 16))
        cp_a.wait()
        accs = _sum_rows(rows_a, zeros)
        cp_b.wait()
        accs = _sum_rows(rows_b, accs)
        for j in range(C // 16):
            outv[pl.ds(e * C + 16 * j, 16)] = accs[j]

    pltpu.sync_copy(outv, out_hbm.at[pl.ds(wid * EPW * C, EPW * C)])


def kernel(x, y, embed, W, b):
    wstack = jnp.stack([
        jnp.transpose(W[:, :D]) / LX,
        jnp.transpose(W[:, D:]) / LY,
    ])
    bias2d = b.reshape(1, C).astype(jnp.float32)
    table = _build_table(embed, wstack, bias2d)

    pad = jnp.full((B, 6), ZERO_ROW, dtype=jnp.int32).at[:, 0].set(BIAS_ROW)
    idx_all = jnp.concatenate(
        [x.astype(jnp.int32), y.astype(jnp.int32) + V, pad], axis=1
    ).reshape(-1)

    out = _sc_pool(table, idx_all)
    return out.reshape(B, C)
